# Initial kernel scaffold; baseline (speedup 1.0000x reference)
import jax
import jax.numpy as jnp
from jax.experimental import pallas as pl


def kernel(src, emb, w_ih_f, w_hh_f, b_ih_f, b_hh_f, w_ih_b, w_hh_b, b_ih_b, b_hh_b):
    raise NotImplementedError("write your pallas kernel here")


# speedup vs baseline: 3.6106x; 3.6106x over previous
import functools

import jax
import jax.numpy as jnp
from jax import lax
from jax.experimental import pallas as pl
from jax.experimental.pallas import tpu as pltpu
from jax.experimental.pallas import tpu_sc as plsc

VOCAB = 100000
EMB = 64
HID = 128
SEQ = 200
BATCH = 1024

_NW = 32
_TOK = SEQ * BATCH
_BPW = _TOK // _NW
_CH = 128
_NCH = _BPW // _CH


def _sc_gather(table, idx3d):
    mesh = plsc.VectorSubcoreMesh(core_axis_name="c", subcore_axis_name="s")

    @functools.partial(
        pl.kernel,
        mesh=mesh,
        out_type=jax.ShapeDtypeStruct((_TOK, EMB), jnp.float32),
        scratch_types=[
            pltpu.VMEM((_NCH, _CH), jnp.int32),
            pltpu.VMEM((_CH, EMB), jnp.float32),
            pltpu.VMEM((_CH, EMB), jnp.float32),
            pltpu.SemaphoreType.DMA,
            pltpu.SemaphoreType.DMA,
        ],
        compiler_params=pltpu.CompilerParams(use_tc_tiling_on_sc=False),
    )
    def gather_kernel(table_hbm, idx_hbm, out_hbm, idx_v, buf0, buf1, sem0, sem1):
        wid = lax.axis_index("s") * 2 + lax.axis_index("c")
        base = wid * _BPW
        pltpu.sync_copy(idx_hbm.at[wid], idx_v)

        pltpu.async_copy(table_hbm.at[idx_v.at[0]], buf0, sem0)
        pltpu.async_copy(table_hbm.at[idx_v.at[1]], buf1, sem1)

        def body(g, carry):
            j0 = 2 * g
            pltpu.make_async_copy(table_hbm.at[idx_v.at[0]], buf0, sem0).wait()
            pltpu.sync_copy(buf0, out_hbm.at[pl.ds(base + j0 * _CH, _CH)])

            @pl.when(j0 + 2 < _NCH)
            def _():
                pltpu.async_copy(table_hbm.at[idx_v.at[j0 + 2]], buf0, sem0)

            pltpu.make_async_copy(table_hbm.at[idx_v.at[0]], buf1, sem1).wait()
            pltpu.sync_copy(buf1, out_hbm.at[pl.ds(base + (j0 + 1) * _CH, _CH)])

            @pl.when(j0 + 3 < _NCH)
            def _():
                pltpu.async_copy(table_hbm.at[idx_v.at[j0 + 3]], buf1, sem1)
            return carry

        lax.fori_loop(0, _NCH // 2, body, None)

    return gather_kernel(table, idx3d)


def _gru_body(xf_ref, xb_ref, wif, whf, bif, bhf, wib, whb, bib, bhb,
              out_ref, hf, hb):
    t = pl.program_id(0)

    @pl.when(t == 0)
    def _init():
        hf[...] = jnp.zeros((BATCH, HID), jnp.float32)
        hb[...] = jnp.zeros((BATCH, HID), jnp.float32)

    def step(x, h_ref, wiT, whT, bi, bh):
        h = h_ref[...]
        gi = jnp.dot(x, wiT[...], preferred_element_type=jnp.float32) + bi[...]
        gh = jnp.dot(h, whT[...], preferred_element_type=jnp.float32) + bh[...]
        r = jax.nn.sigmoid(gi[:, :HID] + gh[:, :HID])
        z = jax.nn.sigmoid(gi[:, HID:2 * HID] + gh[:, HID:2 * HID])
        n = jnp.tanh(gi[:, 2 * HID:] + r * gh[:, 2 * HID:])
        h_ref[...] = (1.0 - z) * n + z * h

    step(xf_ref[...], hf, wif, whf, bif, bhf)
    step(xb_ref[...], hb, wib, whb, bib, bhb)

    @pl.when(t == SEQ - 1)
    def _out():
        out_ref[0] = hf[...]
        out_ref[1] = hb[...]


def _tc_gru(embedded, wiT_f, whT_f, bi_f, bh_f, wiT_b, whT_b, bi_b, bh_b):
    w_spec = lambda: pl.BlockSpec((EMB, 3 * HID), lambda t: (0, 0))
    h_spec = lambda: pl.BlockSpec((HID, 3 * HID), lambda t: (0, 0))
    b_spec = lambda: pl.BlockSpec((1, 3 * HID), lambda t: (0, 0))
    return pl.pallas_call(
        _gru_body,
        grid=(SEQ,),
        in_specs=[
            pl.BlockSpec((BATCH, EMB), lambda t: (t, 0)),
            pl.BlockSpec((BATCH, EMB), lambda t: (SEQ - 1 - t, 0)),
            w_spec(), h_spec(), b_spec(), b_spec(),
            w_spec(), h_spec(), b_spec(), b_spec(),
        ],
        out_specs=pl.BlockSpec((2, BATCH, HID), lambda t: (0, 0, 0)),
        out_shape=jax.ShapeDtypeStruct((2, BATCH, HID), jnp.float32),
        scratch_shapes=[
            pltpu.VMEM((BATCH, HID), jnp.float32),
            pltpu.VMEM((BATCH, HID), jnp.float32),
        ],
    )(embedded, embedded, wiT_f, whT_f, bi_f, bh_f, wiT_b, whT_b, bi_b, bh_b)


def kernel(src, emb, w_ih_f, w_hh_f, b_ih_f, b_hh_f, w_ih_b, w_hh_b, b_ih_b, b_hh_b):
    idx3d = src.reshape(_NW, _NCH, _CH)
    embedded = _sc_gather(emb, idx3d)
    return _tc_gru(
        embedded,
        w_ih_f.T, w_hh_f.T, b_ih_f.reshape(1, -1), b_hh_f.reshape(1, -1),
        w_ih_b.T, w_hh_b.T, b_ih_b.reshape(1, -1), b_hh_b.reshape(1, -1),
    )
